# baseline (speedup 1.0000x reference)
import functools

import jax
import jax.numpy as jnp
from jax import lax
from jax.experimental import pallas as pl
from jax.experimental.pallas import tpu as pltpu
from jax.experimental.pallas import tpu_sc as plsc

_B = 128
_F = 10000
_N = _B * 3 * _F
_NW = 32
_PF = 313
_CF = 40
_NCHUNK = 8
_BLK = 3 * _B
_CHUNK = _CF * _BLK

_TINY = 1.1754943508222875e-38
_LN2 = 0.6931471805599453
_SQRT2 = 1.4142135


def _ln(v):
    i = lax.bitcast_convert_type(v, jnp.int32)
    e = lax.shift_right_logical(i, 23) - 127
    m = lax.bitcast_convert_type(
        jnp.bitwise_or(jnp.bitwise_and(i, 0x007FFFFF), 0x3F800000), jnp.float32)
    big = m > _SQRT2
    m = jnp.where(big, m * jnp.float32(0.5), m)
    ef = (e + big.astype(jnp.int32)).astype(jnp.float32)
    t = m - jnp.float32(1.0)
    p = jnp.float32(-0.14166762502629923)
    for c in (0.21813709864972955, -0.25364295960059036, 0.3327619122985541,
              -0.49992316617099586, 1.000002865808644):
        p = p * t + jnp.float32(c)
    return ef * jnp.float32(_LN2) + t * p


_MESH = plsc.VectorSubcoreMesh(core_axis_name="c", subcore_axis_name="s")


@functools.partial(
    pl.kernel,
    mesh=_MESH,
    compiler_params=pltpu.CompilerParams(needs_layout_passes=False),
    out_type=jax.ShapeDtypeStruct((_N,), jnp.float32),
    scratch_types=(
        [pltpu.VMEM((_CHUNK,), jnp.float32)] * 6
        + [pltpu.SemaphoreType.DMA] * 6
    ),
)
def _sc_gumbel_softmax(x_hbm, u_hbm, out_hbm,
                       xb0, xb1, ub0, ub1, ob0, ob1,
                       sx0, sx1, su0, su1, so0, so1):
    wid = lax.axis_index("s") * 2 + lax.axis_index("c")
    f0 = jnp.minimum(wid * _PF, _F - _PF)
    xbs, ubs, obs = (xb0, xb1), (ub0, ub1), (ob0, ob1)
    sxs, sus, sos = (sx0, sx1), (su0, su1), (so0, so1)

    def chunk_off(ci):
        return (f0 + min(ci * _CF, _PF - _CF)) * _BLK

    def start_in(ci):
        s = ci & 1
        off = chunk_off(ci)
        cx = pltpu.async_copy(x_hbm.at[pl.ds(off, _CHUNK)], xbs[s], sxs[s])
        cu = pltpu.async_copy(u_hbm.at[pl.ds(off, _CHUNK)], ubs[s], sus[s])
        return cx, cu

    def compute(ci):
        s = ci & 1
        xb, ub, ob = xbs[s], ubs[s], obs[s]

        @plsc.parallel_loop(0, _CF * 8, unroll=4)
        def vec(j):
            o0 = lax.shift_right_logical(j, 3) * _BLK + jnp.bitwise_and(j, 7) * 16
            o1 = o0 + _B
            o2 = o1 + _B
            x0 = xb[pl.ds(o0, 16)]
            x1 = xb[pl.ds(o1, 16)]
            x2 = xb[pl.ds(o2, 16)]
            u0 = ub[pl.ds(o0, 16)]
            u1 = ub[pl.ds(o1, 16)]
            u2 = ub[pl.ds(o2, 16)]
            tiny = jnp.float32(_TINY)
            l0 = _ln(jnp.maximum(u0, tiny))
            l1 = _ln(jnp.maximum(u1, tiny))
            l2 = _ln(jnp.maximum(u2, tiny))
            a = l0 * l0
            b = l1 * l1
            c = l2 * l2
            two = jnp.float32(2.0)
            e0 = jnp.exp(x0 * two) * (b * c)
            e1 = jnp.exp(x1 * two) * (a * c)
            e2 = jnp.exp(x2 * two) * (a * b)
            r = jnp.float32(1.0) / (e0 + e1 + e2)
            ob[pl.ds(o0, 16)] = e0 * r
            ob[pl.ds(o1, 16)] = e1 * r
            ob[pl.ds(o2, 16)] = e2 * r

        return pltpu.async_copy(ob, out_hbm.at[pl.ds(chunk_off(ci), _CHUNK)],
                                sos[s])

    in_flight = {0: start_in(0)}
    out_flight = {}
    for ci in range(_NCHUNK):
        if ci + 1 < _NCHUNK:
            in_flight[ci + 1] = start_in(ci + 1)
        for cpy in in_flight.pop(ci):
            cpy.wait()
        if ci - 2 in out_flight:
            out_flight.pop(ci - 2).wait()
        out_flight[ci] = compute(ci)
    for ci, cpy in out_flight.items():
        cpy.wait()


def kernel(logits, uniform):
    x = logits.T.reshape(_N)
    u = uniform.transpose(1, 2, 3, 0).reshape(_N)
    out = _sc_gumbel_softmax(x, u)
    return out.reshape(3 * _F, _B).T


# speedup vs baseline: 1.2183x; 1.2183x over previous
import functools

import jax
import jax.numpy as jnp
from jax import lax
from jax.experimental import pallas as pl
from jax.experimental.pallas import tpu as pltpu
from jax.experimental.pallas import tpu_sc as plsc

_B = 128
_F = 10000
_N = _B * 3 * _F
_NW = 32
_PF = 313
_CF = 40
_NCHUNK = 8
_BLK = 3 * _B
_CHUNK = _CF * _BLK

_TINY = 1.1754943508222875e-38
_LN2 = 0.6931471805599453
_SQRT2 = 1.4142135


def _ln(v):
    i = lax.bitcast_convert_type(v, jnp.int32)
    e = lax.shift_right_arithmetic(i - 0x3F3504F3, 23)
    m = lax.bitcast_convert_type(i - lax.shift_left(e, 23), jnp.float32)
    ef = e.astype(jnp.float32)
    t = m - jnp.float32(1.0)
    p = jnp.float32(-0.14166762502629923)
    for c in (0.21813709864972955, -0.25364295960059036, 0.3327619122985541,
              -0.49992316617099586, 1.000002865808644):
        p = p * t + jnp.float32(c)
    return ef * jnp.float32(_LN2) + t * p


_MESH = plsc.VectorSubcoreMesh(core_axis_name="c", subcore_axis_name="s")


@functools.partial(
    pl.kernel,
    mesh=_MESH,
    compiler_params=pltpu.CompilerParams(needs_layout_passes=False),
    out_type=jax.ShapeDtypeStruct((_N,), jnp.float32),
    scratch_types=(
        [pltpu.VMEM((_CHUNK,), jnp.float32)] * 6
        + [pltpu.SemaphoreType.DMA] * 6
    ),
)
def _sc_gumbel_softmax(x_hbm, u_hbm, out_hbm,
                       xb0, xb1, ub0, ub1, ob0, ob1,
                       sx0, sx1, su0, su1, so0, so1):
    wid = lax.axis_index("s") * 2 + lax.axis_index("c")
    f0 = jnp.minimum(wid * _PF, _F - _PF)
    xbs, ubs, obs = (xb0, xb1), (ub0, ub1), (ob0, ob1)
    sxs, sus, sos = (sx0, sx1), (su0, su1), (so0, so1)

    def chunk_off(ci):
        return (f0 + min(ci * _CF, _PF - _CF)) * _BLK

    def start_in(ci):
        s = ci & 1
        off = chunk_off(ci)
        cx = pltpu.async_copy(x_hbm.at[pl.ds(off, _CHUNK)], xbs[s], sxs[s])
        cu = pltpu.async_copy(u_hbm.at[pl.ds(off, _CHUNK)], ubs[s], sus[s])
        return cx, cu

    def compute(ci):
        s = ci & 1
        xb, ub, ob = xbs[s], ubs[s], obs[s]

        def vec(j, carry):
            o0 = lax.shift_right_logical(j, 3) * _BLK + jnp.bitwise_and(j, 7) * 16
            o1 = o0 + _B
            o2 = o1 + _B
            x0 = xb[pl.ds(o0, 16)]
            x1 = xb[pl.ds(o1, 16)]
            x2 = xb[pl.ds(o2, 16)]
            u0 = ub[pl.ds(o0, 16)]
            u1 = ub[pl.ds(o1, 16)]
            u2 = ub[pl.ds(o2, 16)]
            tiny = jnp.float32(_TINY)
            l0 = _ln(jnp.maximum(u0, tiny))
            l1 = _ln(jnp.maximum(u1, tiny))
            l2 = _ln(jnp.maximum(u2, tiny))
            a = l0 * l0
            b = l1 * l1
            c = l2 * l2
            two = jnp.float32(2.0)
            e0 = jnp.exp(x0 * two) * (b * c)
            e1 = jnp.exp(x1 * two) * (a * c)
            e2 = jnp.exp(x2 * two) * (a * b)
            r = jnp.float32(1.0) / (e0 + e1 + e2)
            ob[pl.ds(o0, 16)] = e0 * r
            ob[pl.ds(o1, 16)] = e1 * r
            ob[pl.ds(o2, 16)] = e2 * r
            return carry

        lax.fori_loop(0, _CF * 8, vec, 0, unroll=False)
        return pltpu.async_copy(ob, out_hbm.at[pl.ds(chunk_off(ci), _CHUNK)],
                                sos[s])

    in_flight = {0: start_in(0)}
    out_flight = {}
    for ci in range(_NCHUNK):
        if ci + 1 < _NCHUNK:
            in_flight[ci + 1] = start_in(ci + 1)
        for cpy in in_flight.pop(ci):
            cpy.wait()
        if ci - 2 in out_flight:
            out_flight.pop(ci - 2).wait()
        out_flight[ci] = compute(ci)
    for ci, cpy in out_flight.items():
        cpy.wait()


def kernel(logits, uniform):
    x = logits.T.reshape(_N)
    u = uniform.transpose(1, 2, 3, 0).reshape(_N)
    out = _sc_gumbel_softmax(x, u)
    return out.reshape(3 * _F, _B).T


# speedup vs baseline: 1.2558x; 1.0308x over previous
import functools

import jax
import jax.numpy as jnp
from jax import lax
from jax.experimental import pallas as pl
from jax.experimental.pallas import tpu as pltpu
from jax.experimental.pallas import tpu_sc as plsc

_B = 128
_F = 10000
_N = _B * 3 * _F
_NW = 32
_PF = 313
_CF = 40
_NCHUNK = 8
_BLK = 3 * _B
_CHUNK = _CF * _BLK

_TINY = 1.1754943508222875e-38
_LN2 = 0.6931471805599453
_SQRT2 = 1.4142135


def _ln(v):
    i = lax.bitcast_convert_type(v, jnp.int32)
    e = lax.shift_right_arithmetic(i - 0x3F3504F3, 23)
    m = lax.bitcast_convert_type(i - lax.shift_left(e, 23), jnp.float32)
    ef = e.astype(jnp.float32)
    t = m - jnp.float32(1.0)
    p = jnp.float32(-0.14166762502629923)
    for c in (0.21813709864972955, -0.25364295960059036, 0.3327619122985541,
              -0.49992316617099586, 1.000002865808644):
        p = p * t + jnp.float32(c)
    return ef * jnp.float32(_LN2) + t * p


_MESH = plsc.VectorSubcoreMesh(core_axis_name="c", subcore_axis_name="s")


@functools.partial(
    pl.kernel,
    mesh=_MESH,
    compiler_params=pltpu.CompilerParams(needs_layout_passes=False),
    out_type=jax.ShapeDtypeStruct((_N,), jnp.float32),
    scratch_types=(
        [pltpu.VMEM((_CHUNK,), jnp.float32)] * 6
        + [pltpu.SemaphoreType.DMA] * 6
    ),
)
def _sc_gumbel_softmax(x_hbm, u_hbm, out_hbm,
                       xb0, xb1, ub0, ub1, ob0, ob1,
                       sx0, sx1, su0, su1, so0, so1):
    wid = lax.axis_index("s") * 2 + lax.axis_index("c")
    f0 = jnp.minimum(wid * _PF, _F - _PF)
    xbs, ubs, obs = (xb0, xb1), (ub0, ub1), (ob0, ob1)
    sxs, sus, sos = (sx0, sx1), (su0, su1), (so0, so1)

    def chunk_off(ci):
        return (f0 + min(ci * _CF, _PF - _CF)) * _BLK

    def start_in(ci):
        s = ci & 1
        off = chunk_off(ci)
        cx = pltpu.async_copy(x_hbm.at[pl.ds(off, _CHUNK)], xbs[s], sxs[s])
        cu = pltpu.async_copy(u_hbm.at[pl.ds(off, _CHUNK)], ubs[s], sus[s])
        return cx, cu

    def compute(ci):
        s = ci & 1
        xb, ub, ob = xbs[s], ubs[s], obs[s]

        def vec(j, carry):
            o0 = lax.shift_right_logical(j, 3) * _BLK + jnp.bitwise_and(j, 7) * 16
            o1 = o0 + _B
            o2 = o1 + _B
            x0 = xb[pl.ds(o0, 16)]
            x1 = xb[pl.ds(o1, 16)]
            x2 = xb[pl.ds(o2, 16)]
            u0 = ub[pl.ds(o0, 16)]
            u1 = ub[pl.ds(o1, 16)]
            u2 = ub[pl.ds(o2, 16)]
            l0 = _ln(u0)
            l1 = _ln(u1)
            l2 = _ln(u2)
            a = l0 * l0
            b = l1 * l1
            c = l2 * l2
            two = jnp.float32(2.0)
            e0 = jnp.exp(x0 * two) * (b * c)
            e1 = jnp.exp(x1 * two) * (a * c)
            e2 = jnp.exp(x2 * two) * (a * b)
            r = jnp.float32(1.0) / (e0 + e1 + e2)
            ob[pl.ds(o0, 16)] = e0 * r
            ob[pl.ds(o1, 16)] = e1 * r
            ob[pl.ds(o2, 16)] = e2 * r
            return carry

        lax.fori_loop(0, _CF * 8, vec, 0, unroll=False)
        return pltpu.async_copy(ob, out_hbm.at[pl.ds(chunk_off(ci), _CHUNK)],
                                sos[s])

    in_flight = {0: start_in(0)}
    out_flight = {}
    for ci in range(_NCHUNK):
        if ci + 1 < _NCHUNK:
            in_flight[ci + 1] = start_in(ci + 1)
        for cpy in in_flight.pop(ci):
            cpy.wait()
        if ci - 2 in out_flight:
            out_flight.pop(ci - 2).wait()
        out_flight[ci] = compute(ci)
    for ci, cpy in out_flight.items():
        cpy.wait()


def kernel(logits, uniform):
    x = logits.T.reshape(_N)
    u = uniform.transpose(1, 2, 3, 0).reshape(_N)
    out = _sc_gumbel_softmax(x, u)
    return out.reshape(3 * _F, _B).T


# speedup vs baseline: 1.2720x; 1.0129x over previous
import functools

import jax
import jax.numpy as jnp
from jax import lax
from jax.experimental import pallas as pl
from jax.experimental.pallas import tpu as pltpu
from jax.experimental.pallas import tpu_sc as plsc

_B = 128
_F = 10000
_N = _B * 3 * _F
_NW = 32
_PF = 313
_CF = 48
_NCHUNK = 7
_BLK = 3 * _B
_CHUNK = _CF * _BLK

_TINY = 1.1754943508222875e-38
_LN2 = 0.6931471805599453
_SQRT2 = 1.4142135


def _ln(v):
    i = lax.bitcast_convert_type(v, jnp.int32)
    e = lax.shift_right_arithmetic(i - 0x3F3504F3, 23)
    m = lax.bitcast_convert_type(i - lax.shift_left(e, 23), jnp.float32)
    ef = e.astype(jnp.float32)
    t = m - jnp.float32(1.0)
    p = jnp.float32(0.17516918630105072)
    for c in (-0.2681063335014877, 0.33602639315620575,
              -0.4996037331116914, 0.9999751705497111):
        p = p * t + jnp.float32(c)
    return ef * jnp.float32(_LN2) + t * p


_MESH = plsc.VectorSubcoreMesh(core_axis_name="c", subcore_axis_name="s")


@functools.partial(
    pl.kernel,
    mesh=_MESH,
    compiler_params=pltpu.CompilerParams(needs_layout_passes=False),
    out_type=jax.ShapeDtypeStruct((_N,), jnp.float32),
    scratch_types=(
        [pltpu.VMEM((_CHUNK,), jnp.float32)] * 6
        + [pltpu.SemaphoreType.DMA] * 6
    ),
)
def _sc_gumbel_softmax(x_hbm, u_hbm, out_hbm,
                       xb0, xb1, ub0, ub1, ob0, ob1,
                       sx0, sx1, su0, su1, so0, so1):
    wid = lax.axis_index("s") * 2 + lax.axis_index("c")
    f0 = jnp.minimum(wid * _PF, _F - _PF)
    xbs, ubs, obs = (xb0, xb1), (ub0, ub1), (ob0, ob1)
    sxs, sus, sos = (sx0, sx1), (su0, su1), (so0, so1)

    def chunk_off(ci):
        return (f0 + min(ci * _CF, _PF - _CF)) * _BLK

    def start_in(ci):
        s = ci & 1
        off = chunk_off(ci)
        cx = pltpu.async_copy(x_hbm.at[pl.ds(off, _CHUNK)], xbs[s], sxs[s])
        cu = pltpu.async_copy(u_hbm.at[pl.ds(off, _CHUNK)], ubs[s], sus[s])
        return cx, cu

    def compute(ci):
        s = ci & 1
        xb, ub, ob = xbs[s], ubs[s], obs[s]

        def vec(j, carry):
            o0 = lax.shift_right_logical(j, 3) * _BLK + jnp.bitwise_and(j, 7) * 16
            o1 = o0 + _B
            o2 = o1 + _B
            x0 = xb[pl.ds(o0, 16)]
            x1 = xb[pl.ds(o1, 16)]
            x2 = xb[pl.ds(o2, 16)]
            u0 = ub[pl.ds(o0, 16)]
            u1 = ub[pl.ds(o1, 16)]
            u2 = ub[pl.ds(o2, 16)]
            l0 = _ln(u0)
            l1 = _ln(u1)
            l2 = _ln(u2)
            a = l0 * l0
            b = l1 * l1
            c = l2 * l2
            two = jnp.float32(2.0)
            e0 = jnp.exp(x0 * two) * (b * c)
            e1 = jnp.exp(x1 * two) * (a * c)
            e2 = jnp.exp(x2 * two) * (a * b)
            r = jnp.float32(1.0) / (e0 + e1 + e2)
            ob[pl.ds(o0, 16)] = e0 * r
            ob[pl.ds(o1, 16)] = e1 * r
            ob[pl.ds(o2, 16)] = e2 * r
            return carry

        lax.fori_loop(0, _CF * 8, vec, 0, unroll=False)
        return pltpu.async_copy(ob, out_hbm.at[pl.ds(chunk_off(ci), _CHUNK)],
                                sos[s])

    in_flight = {0: start_in(0)}
    out_flight = {}
    for ci in range(_NCHUNK):
        if ci + 1 < _NCHUNK:
            in_flight[ci + 1] = start_in(ci + 1)
        for cpy in in_flight.pop(ci):
            cpy.wait()
        if ci - 2 in out_flight:
            out_flight.pop(ci - 2).wait()
        out_flight[ci] = compute(ci)
    for ci, cpy in out_flight.items():
        cpy.wait()


def kernel(logits, uniform):
    x = logits.T.reshape(_N)
    u = uniform.transpose(1, 2, 3, 0).reshape(_N)
    out = _sc_gumbel_softmax(x, u)
    return out.reshape(3 * _F, _B).T


# speedup vs baseline: 1.3050x; 1.0259x over previous
import functools

import jax
import jax.numpy as jnp
from jax import lax
from jax.experimental import pallas as pl
from jax.experimental.pallas import tpu as pltpu
from jax.experimental.pallas import tpu_sc as plsc

_B = 128
_F = 10000
_N = _B * 3 * _F
_NW = 32
_PF = 313
_CF = 48
_NCHUNK = 7
_BLK = 3 * _B
_CHUNK = _CF * _BLK

_TINY = 1.1754943508222875e-38
_LN2 = 0.6931471805599453
_SQRT2 = 1.4142135


def _ln(v):
    i = lax.bitcast_convert_type(v, jnp.int32)
    e = lax.shift_right_arithmetic(i - 0x3F3504F3, 23)
    m = lax.bitcast_convert_type(i - lax.shift_left(e, 23), jnp.float32)
    ef = e.astype(jnp.float32)
    t = m - jnp.float32(1.0)
    p = jnp.float32(0.2527157163930808)
    for c in (-0.38679567777351886, 0.48478361101426315,
              -0.7207758281698503, 1.4426592195641637):
        p = p * t + jnp.float32(c)
    return ef + t * p


_MESH = plsc.VectorSubcoreMesh(core_axis_name="c", subcore_axis_name="s")


@functools.partial(
    pl.kernel,
    mesh=_MESH,
    compiler_params=pltpu.CompilerParams(needs_layout_passes=False),
    out_type=jax.ShapeDtypeStruct((_N,), jnp.float32),
    scratch_types=(
        [pltpu.VMEM((_CHUNK,), jnp.float32)] * 6
        + [pltpu.SemaphoreType.DMA] * 6
    ),
)
def _sc_gumbel_softmax(x_hbm, u_hbm, out_hbm,
                       xb0, xb1, ub0, ub1, ob0, ob1,
                       sx0, sx1, su0, su1, so0, so1):
    wid = lax.axis_index("s") * 2 + lax.axis_index("c")
    f0 = jnp.minimum(wid * _PF, _F - _PF)
    xbs, ubs, obs = (xb0, xb1), (ub0, ub1), (ob0, ob1)
    sxs, sus, sos = (sx0, sx1), (su0, su1), (so0, so1)

    def chunk_off(ci):
        return (f0 + min(ci * _CF, _PF - _CF)) * _BLK

    def start_in(ci):
        s = ci & 1
        off = chunk_off(ci)
        cx = pltpu.async_copy(x_hbm.at[pl.ds(off, _CHUNK)], xbs[s], sxs[s])
        cu = pltpu.async_copy(u_hbm.at[pl.ds(off, _CHUNK)], ubs[s], sus[s])
        return cx, cu

    def compute(ci):
        s = ci & 1
        xb, ub, ob = xbs[s], ubs[s], obs[s]

        def vec(j, carry):
            o0 = lax.shift_right_logical(j, 3) * _BLK + jnp.bitwise_and(j, 7) * 16
            o1 = o0 + _B
            o2 = o1 + _B
            x0 = xb[pl.ds(o0, 16)]
            x1 = xb[pl.ds(o1, 16)]
            x2 = xb[pl.ds(o2, 16)]
            u0 = ub[pl.ds(o0, 16)]
            u1 = ub[pl.ds(o1, 16)]
            u2 = ub[pl.ds(o2, 16)]
            l0 = _ln(u0)
            l1 = _ln(u1)
            l2 = _ln(u2)
            a = l0 * l0
            b = l1 * l1
            c = l2 * l2
            two = jnp.float32(2.0)
            e0 = jnp.exp(x0 * two) * (b * c)
            e1 = jnp.exp(x1 * two) * (a * c)
            e2 = jnp.exp(x2 * two) * (a * b)
            r = jnp.float32(1.0) / (e0 + e1 + e2)
            ob[pl.ds(o0, 16)] = e0 * r
            ob[pl.ds(o1, 16)] = e1 * r
            ob[pl.ds(o2, 16)] = e2 * r
            return carry

        lax.fori_loop(0, _CF * 8, vec, 0, unroll=False)
        return pltpu.async_copy(ob, out_hbm.at[pl.ds(chunk_off(ci), _CHUNK)],
                                sos[s])

    in_flight = {0: start_in(0)}
    out_flight = {}
    for ci in range(_NCHUNK):
        if ci + 1 < _NCHUNK:
            in_flight[ci + 1] = start_in(ci + 1)
        for cpy in in_flight.pop(ci):
            cpy.wait()
        if ci - 2 in out_flight:
            out_flight.pop(ci - 2).wait()
        out_flight[ci] = compute(ci)
    for ci, cpy in out_flight.items():
        cpy.wait()


def kernel(logits, uniform):
    x = logits.T.reshape(_N)
    u = uniform.transpose(1, 2, 3, 0).reshape(_N)
    out = _sc_gumbel_softmax(x, u)
    return out.reshape(3 * _F, _B).T


# speedup vs baseline: 2.6146x; 2.0035x over previous
import functools

import jax
import jax.numpy as jnp
from jax import lax
from jax.experimental import pallas as pl
from jax.experimental.pallas import tpu as pltpu
from jax.experimental.pallas import tpu_sc as plsc

_B = 128
_F = 10000
_N = _B * 3 * _F
_NW = 32
_PF = 313
_CF = 48
_NCHUNK = 7
_BLK = 3 * _B
_CHUNK = _CF * _BLK

_TINY = 1.1754943508222875e-38
_LN2 = 0.6931471805599453
_SQRT2 = 1.4142135


def _ln(v):
    i = lax.bitcast_convert_type(v, jnp.int32)
    e = lax.shift_right_arithmetic(i - 0x3F3504F3, 23)
    m = lax.bitcast_convert_type(i - lax.shift_left(e, 23), jnp.float32)
    ef = e.astype(jnp.float32)
    t = m - jnp.float32(1.0)
    p = jnp.float32(0.2527157163930808)
    for c in (-0.38679567777351886, 0.48478361101426315,
              -0.7207758281698503, 1.4426592195641637):
        p = p * t + jnp.float32(c)
    return ef + t * p


_MESH = plsc.VectorSubcoreMesh(core_axis_name="c", subcore_axis_name="s")


@functools.partial(
    pl.kernel,
    mesh=_MESH,
    compiler_params=pltpu.CompilerParams(needs_layout_passes=False),
    out_type=jax.ShapeDtypeStruct((_N,), jnp.float32),
    scratch_types=(
        [pltpu.VMEM((_CHUNK,), jnp.float32)] * 6
        + [pltpu.SemaphoreType.DMA] * 6
    ),
)
def _sc_gumbel_softmax(x_hbm, u_hbm, out_hbm,
                       xb0, xb1, ub0, ub1, ob0, ob1,
                       sx0, sx1, su0, su1, so0, so1):
    wid = lax.axis_index("s") * 2 + lax.axis_index("c")
    f0 = jnp.minimum(wid * _PF, _F - _PF)
    xbs, ubs, obs = (xb0, xb1), (ub0, ub1), (ob0, ob1)
    sxs, sus, sos = (sx0, sx1), (su0, su1), (so0, so1)

    def chunk_off(ci):
        return (f0 + min(ci * _CF, _PF - _CF)) * _BLK

    def start_in(ci):
        s = ci & 1
        off = chunk_off(ci)
        cx = pltpu.async_copy(x_hbm.at[pl.ds(off, _CHUNK)], xbs[s], sxs[s])
        cu = pltpu.async_copy(u_hbm.at[pl.ds(off, _CHUNK)], ubs[s], sus[s])
        return cx, cu

    def compute(ci):
        s = ci & 1
        xb, ub, ob = xbs[s], ubs[s], obs[s]

        def vec(j, carry):
            o0 = lax.shift_right_logical(j, 3) * _BLK + jnp.bitwise_and(j, 7) * 16
            o1 = o0 + _B
            o2 = o1 + _B
            x0 = xb[pl.ds(o0, 16)]
            x1 = xb[pl.ds(o1, 16)]
            x2 = xb[pl.ds(o2, 16)]
            u0 = ub[pl.ds(o0, 16)]
            u1 = ub[pl.ds(o1, 16)]
            u2 = ub[pl.ds(o2, 16)]
            l0 = _ln(u0)
            l1 = _ln(u1)
            l2 = _ln(u2)
            a = l0 * l0
            b = l1 * l1
            c = l2 * l2
            two = jnp.float32(2.0)
            e0 = jnp.exp(x0 * two) * (b * c)
            e1 = jnp.exp(x1 * two) * (a * c)
            e2 = jnp.exp(x2 * two) * (a * b)
            r = jnp.float32(1.0) / (e0 + e1 + e2)
            ob[pl.ds(o0, 16)] = e0 * r
            ob[pl.ds(o1, 16)] = e1 * r
            ob[pl.ds(o2, 16)] = e2 * r
            return carry

        lax.fori_loop(0, _CF * 8, vec, 0, unroll=False)
        return pltpu.async_copy(ob, out_hbm.at[pl.ds(chunk_off(ci), _CHUNK)],
                                sos[s])

    in_flight = {0: start_in(0)}
    out_flight = {}
    for ci in range(_NCHUNK):
        if ci + 1 < _NCHUNK:
            in_flight[ci + 1] = start_in(ci + 1)
        for cpy in in_flight.pop(ci):
            cpy.wait()
        if ci - 2 in out_flight:
            out_flight.pop(ci - 2).wait()
        out_flight[ci] = compute(ci)
    for ci, cpy in out_flight.items():
        cpy.wait()


_RB = 1200
_RT = 3 * _F


def _tc_body(x_ref, u_ref, o_ref):
    x = x_ref[...]
    u = jnp.clip(u_ref[...], jnp.float32(_TINY), jnp.float32(1.0))
    ll = jnp.log(u)
    w = jnp.exp(x * jnp.float32(2.0)) / (ll * ll)
    a = w + jnp.roll(w, -1, axis=0) + jnp.roll(w, -2, axis=0)
    g = jax.lax.broadcasted_iota(jnp.int32, (_RB, _B), 0) % 3
    s = jnp.where(g == 0, a, jnp.where(g == 1, jnp.roll(a, 1, axis=0),
                                       jnp.roll(a, 2, axis=0)))
    o_ref[...] = w / s


def _tc_gumbel_softmax(xt, ut):
    return pl.pallas_call(
        _tc_body,
        grid=(_RT // _RB,),
        in_specs=[pl.BlockSpec((_RB, _B), lambda i: (i, 0)),
                  pl.BlockSpec((_RB, _B), lambda i: (i, 0))],
        out_specs=pl.BlockSpec((_RB, _B), lambda i: (i, 0)),
        out_shape=jax.ShapeDtypeStruct((_RT, _B), jnp.float32),
    )(xt, ut)


def kernel(logits, uniform):
    xt = logits.T
    ut = uniform.transpose(1, 2, 3, 0).reshape(3 * _F, _B)
    out = _tc_gumbel_softmax(xt, ut)
    return out.T
